# baseline (speedup 1.0000x reference)
import functools

import jax
import jax.numpy as jnp
from jax import lax
from jax.experimental import pallas as pl
from jax.experimental.pallas import tpu as pltpu
from jax.experimental.pallas import tpu_sc as plsc

BATCH = 16384
D = 32
NC = 2
NS = 16
NW = NC * NS
BPW = BATCH // NW
EPW = BPW * D
GCH = 512
NGCH = EPW // GCH


def _gather_body(u_ids, i_ids, u_flat, i_flat, u_out, i_out,
                 idx_v, eidx_u, eidx_i, dst_u, dst_i, sem_u, sem_i):
    wid = lax.axis_index("s") * NC + lax.axis_index("c")
    base = wid * BPW
    pltpu.sync_copy(u_ids.at[pl.ds(base, BPW)], idx_v.at[0])
    pltpu.sync_copy(i_ids.at[pl.ds(base, BPW)], idx_v.at[1])

    lane = lax.iota(jnp.int32, 16)

    def build(tab, eidx):
        def body(g, _):
            vec = idx_v[tab, pl.ds(g * 16, 16)] * 32
            for l in range(16):
                e0 = vec[l] + lane
                p = (g * 16 + l) * 32
                eidx[pl.ds(p, 16)] = e0
                eidx[pl.ds(p + 16, 16)] = e0 + 16
            return _

        lax.fori_loop(0, BPW // 16, body, None)

    build(0, eidx_u)
    build(1, eidx_i)

    copies = []
    for ch in range(NGCH):
        sl = pl.ds(ch * GCH, GCH)
        copies.append(pltpu.async_copy(u_flat.at[eidx_u.at[sl]],
                                       dst_u.at[sl], sem_u))
        copies.append(pltpu.async_copy(i_flat.at[eidx_i.at[sl]],
                                       dst_i.at[sl], sem_i))
    for c in copies:
        c.wait()
    pltpu.sync_copy(dst_u, u_out.at[pl.ds(base * D, EPW)])
    pltpu.sync_copy(dst_i, i_out.at[pl.ds(base * D, EPW)])


_sc_gather = pl.kernel(
    _gather_body,
    out_type=(
        jax.ShapeDtypeStruct((BATCH * D,), jnp.float32),
        jax.ShapeDtypeStruct((BATCH * D,), jnp.float32),
    ),
    mesh=plsc.VectorSubcoreMesh(core_axis_name="c", subcore_axis_name="s"),
    scratch_types=[
        pltpu.VMEM((2, BPW), jnp.int32),
        pltpu.VMEM((EPW,), jnp.int32),
        pltpu.VMEM((EPW,), jnp.int32),
        pltpu.VMEM((EPW,), jnp.float32),
        pltpu.VMEM((EPW,), jnp.float32),
        pltpu.SemaphoreType.DMA,
        pltpu.SemaphoreType.DMA,
    ],
    compiler_params=pltpu.CompilerParams(use_tc_tiling_on_sc=False),
)


def _mlp_body(u_ref, i_ref, w1u_ref, w1i_ref, b1_ref, w2_ref, b2_ref,
              w3_ref, b3_ref, out_ref):
    u = u_ref[...]
    i = i_ref[...]
    h = jnp.dot(u, w1u_ref[...], preferred_element_type=jnp.float32)
    h = h + jnp.dot(i, w1i_ref[...], preferred_element_type=jnp.float32)
    h = jnp.maximum(h + b1_ref[...], 0.0)
    h2 = jnp.dot(h, w2_ref[...], preferred_element_type=jnp.float32)
    h2 = jnp.maximum(h2 + b2_ref[...], 0.0)
    out_ref[...] = jnp.sum(h2 * w3_ref[...], axis=1, keepdims=True) + b3_ref[...]


_mlp = pl.pallas_call(
    _mlp_body,
    out_shape=jax.ShapeDtypeStruct((BATCH, 1), jnp.float32),
)


def kernel(U_ids, I_ids, user_table, item_table, W1, b1, W2, b2, W3, b3):
    u_ids = U_ids.astype(jnp.int32)
    i_ids = I_ids.astype(jnp.int32)
    u_emb_f, i_emb_f = _sc_gather(u_ids, i_ids, user_table.reshape(-1),
                                  item_table.reshape(-1))
    u_emb = u_emb_f.reshape(BATCH, D)
    i_emb = i_emb_f.reshape(BATCH, D)
    return _mlp(u_emb, i_emb, W1[:D], W1[D:], b1.reshape(1, D),
                W2, b2.reshape(1, 8), W3.reshape(1, 8), b3.reshape(1, 1))
